# Initial kernel scaffold; baseline (speedup 1.0000x reference)
import jax
import jax.numpy as jnp
from jax.experimental import pallas as pl


def kernel(x, input_perm):
    raise NotImplementedError("write your pallas kernel here")


# speedup vs baseline: 47.7179x; 47.7179x over previous
import functools
import jax
import jax.numpy as jnp
from jax import lax
from jax.experimental import pallas as pl
from jax.experimental.pallas import tpu as pltpu
from jax.experimental.pallas import tpu_sc as plsc

NC = 2
NS = 16
NW = NC * NS

V = 1_000_000
C1 = 31_360
R1 = C1 // 128
PV = NW * C1

XR = 800
RC = 80
NCH = XR // RC

_mesh = plsc.VectorSubcoreMesh(core_axis_name="c", subcore_axis_name="s")


@functools.partial(
    pl.kernel,
    mesh=_mesh,
    out_type=jax.ShapeDtypeStruct((PV,), jnp.int32),
    scratch_types=[
        pltpu.VMEM((R1, 128), jnp.int32),
        pltpu.VMEM((R1, 128), jnp.int32),
        pltpu.SemaphoreType.DMA,
    ],
)
def _invert_kernel(perm3, ramp3, inv, idx_v, val_v, sem):
    wid = lax.axis_index("s") * NC + lax.axis_index("c")
    pltpu.sync_copy(perm3.at[wid], idx_v)
    pltpu.sync_copy(ramp3.at[wid], val_v)

    def fire(j, carry):
        pltpu.async_copy(val_v.at[j], inv.at[idx_v.at[j]], sem)
        return carry

    lax.fori_loop(0, R1, fire, 0)
    pltpu.make_async_copy(perm3.at[wid], val_v, sem).wait()


@functools.partial(
    pl.kernel,
    mesh=_mesh,
    out_type=jax.ShapeDtypeStruct((NW, XR, 128), jnp.int32),
    scratch_types=[
        pltpu.VMEM((RC, 128), jnp.int32),
        pltpu.VMEM((RC, 128), jnp.int32),
        pltpu.VMEM((RC, 128), jnp.int32),
        pltpu.SemaphoreType.DMA,
    ],
)
def _double_gather_kernel(x3, perm_hbm, inv_hbm, out3, x_v, g_v, o_v, sem):
    wid = lax.axis_index("s") * NC + lax.axis_index("c")

    def drain(nbytes_dst):
        pltpu.make_async_copy(x3.at[wid, pl.ds(0, RC)], nbytes_dst, sem).wait()

    for ch in range(NCH):
        pltpu.sync_copy(x3.at[wid, pl.ds(ch * RC, RC)], x_v)

        def g1(j, carry):
            pltpu.async_copy(perm_hbm.at[x_v.at[j]], g_v.at[j], sem)
            return carry

        lax.fori_loop(0, RC, g1, 0)
        drain(g_v)

        def g2(j, carry):
            pltpu.async_copy(inv_hbm.at[g_v.at[j]], o_v.at[j], sem)
            return carry

        lax.fori_loop(0, RC, g2, 0)
        drain(o_v)

        pltpu.sync_copy(o_v, out3.at[wid, pl.ds(ch * RC, RC)])


def kernel(x, input_perm):
    pad = jnp.arange(V, PV, dtype=jnp.int32)
    perm_p = jnp.concatenate([input_perm.astype(jnp.int32), pad])
    ramp = jnp.arange(PV, dtype=jnp.int32)
    inv = _invert_kernel(perm_p.reshape(NW, R1, 128), ramp.reshape(NW, R1, 128))
    x3 = x.reshape(NW, XR, 128)
    out3 = _double_gather_kernel(x3, perm_p, inv)
    return out3.reshape(x.shape)


# speedup vs baseline: 54.2482x; 1.1369x over previous
import functools
import jax
import jax.numpy as jnp
from jax import lax
from jax.experimental import pallas as pl
from jax.experimental.pallas import tpu as pltpu
from jax.experimental.pallas import tpu_sc as plsc

NC = 2
NS = 16
NW = NC * NS

V = 1_000_000
C1 = 31_360
R1 = C1 // 128
PV = NW * C1

XR = 800
RC = 80
NCH = XR // RC

_mesh = plsc.VectorSubcoreMesh(core_axis_name="c", subcore_axis_name="s")


@functools.partial(
    pl.kernel,
    mesh=_mesh,
    out_type=jax.ShapeDtypeStruct((PV,), jnp.int32),
    scratch_types=[
        pltpu.VMEM((R1, 128), jnp.int32),
        pltpu.VMEM((R1, 128), jnp.int32),
        pltpu.SemaphoreType.DMA,
    ],
)
def _invert_kernel(perm3, ramp3, inv, idx_v, val_v, sem):
    wid = lax.axis_index("s") * NC + lax.axis_index("c")
    pltpu.sync_copy(perm3.at[wid], idx_v)
    pltpu.sync_copy(ramp3.at[wid], val_v)

    def fire(j, carry):
        pltpu.async_copy(val_v.at[j], inv.at[idx_v.at[j]], sem)
        return carry

    lax.fori_loop(0, R1, fire, 0)
    pltpu.make_async_copy(perm3.at[wid], val_v, sem).wait()


@functools.partial(
    pl.kernel,
    mesh=_mesh,
    out_type=jax.ShapeDtypeStruct((NW, R1, 128), jnp.int32),
    scratch_types=[
        pltpu.VMEM((R1, 128), jnp.int32),
        pltpu.VMEM((R1, 128), jnp.int32),
        pltpu.SemaphoreType.DMA,
    ],
)
def _compose_kernel(perm3, inv, t3, idx_v, t_v, sem):
    wid = lax.axis_index("s") * NC + lax.axis_index("c")
    pltpu.sync_copy(perm3.at[wid], idx_v)

    def fire(j, carry):
        pltpu.async_copy(inv.at[idx_v.at[j]], t_v.at[j], sem)
        return carry

    lax.fori_loop(0, R1, fire, 0)
    pltpu.make_async_copy(perm3.at[wid], t_v, sem).wait()
    pltpu.sync_copy(t_v, t3.at[wid])


SPC = PV // NS


@functools.partial(
    pl.kernel,
    mesh=_mesh,
    out_type=jax.ShapeDtypeStruct((NW, XR, 128), jnp.int32),
    scratch_types=[
        pltpu.VMEM((RC, 128), jnp.int32),
        pltpu.VMEM((RC, 128), jnp.int32),
        pltpu.VMEM_SHARED((PV,), jnp.int32),
        pltpu.SemaphoreType.DMA,
    ],
)
def _lookup_kernel(x3, t_hbm, out3, x_v, o_v, t_sh, sem):
    sid = lax.axis_index("s")
    wid = sid * NC + lax.axis_index("c")

    pltpu.sync_copy(t_hbm.at[pl.ds(sid * SPC, SPC)],
                    t_sh.at[pl.ds(sid * SPC, SPC)])
    plsc.subcore_barrier()

    for ch in range(NCH):
        pltpu.sync_copy(x3.at[wid, pl.ds(ch * RC, RC)], x_v)

        def g1(j, carry):
            pltpu.async_copy(t_sh.at[x_v.at[j]], o_v.at[j], sem)
            return carry

        lax.fori_loop(0, RC, g1, 0)
        pltpu.make_async_copy(x3.at[wid, pl.ds(0, RC)], o_v, sem).wait()

        pltpu.sync_copy(o_v, out3.at[wid, pl.ds(ch * RC, RC)])


def kernel(x, input_perm):
    pad = jnp.arange(V, PV, dtype=jnp.int32)
    perm_p = jnp.concatenate([input_perm.astype(jnp.int32), pad])
    ramp = jnp.arange(PV, dtype=jnp.int32)
    perm3 = perm_p.reshape(NW, R1, 128)
    inv = _invert_kernel(perm3, ramp.reshape(NW, R1, 128))
    t3 = _compose_kernel(perm3, inv)
    x3 = x.reshape(NW, XR, 128)
    out3 = _lookup_kernel(x3, t3.reshape(PV))
    return out3.reshape(x.shape)


# speedup vs baseline: 341.1008x; 6.2878x over previous
import functools
import jax
import jax.numpy as jnp
from jax import lax
from jax.experimental import pallas as pl
from jax.experimental.pallas import tpu as pltpu
from jax.experimental.pallas import tpu_sc as plsc

NC = 2
NS = 16
NW = NC * NS

V = 1_000_000
C1 = 31_744
R1 = C1 // 128
PV = NW * C1
SPC = PV // NS
RS = SPC // 128
GRP = 16
NGRP = RS // GRP
CG = 8
NCG = R1 // CG

XR = 800
RC = 80
NCH = XR // RC

_mesh = plsc.VectorSubcoreMesh(core_axis_name="c", subcore_axis_name="s")


@functools.partial(
    pl.kernel,
    mesh=_mesh,
    out_type=jax.ShapeDtypeStruct((NW, R1, 128), jnp.int32),
    scratch_types=[
        pltpu.VMEM((RS, 128), jnp.int32),
        pltpu.VMEM((GRP, 128), jnp.int32),
        pltpu.VMEM_SHARED((PV,), jnp.int32),
        pltpu.SemaphoreType.DMA,
    ],
)
def _build_kernel(perm2d, t3, idx_v, val_v, inv_sh, sem):
    cid = lax.axis_index("c")
    sid = lax.axis_index("s")
    wid = sid * NC + cid
    base = sid * SPC
    lane = lax.iota(jnp.int32, 16)

    pltpu.sync_copy(perm2d.at[pl.ds(sid * RS, RS)], idx_v)

    def grp(g, carry):
        def mkval(t, c2):
            val_v[t // 8, pl.ds((t % 8) * 16, 16)] = lane + (
                base + g * (GRP * 128) + t * 16)
            return c2

        lax.fori_loop(0, GRP * 8, mkval, 0)

        def fire(j, c2):
            pltpu.async_copy(val_v.at[j], inv_sh.at[idx_v.at[g * GRP + j]],
                             sem)
            return c2

        lax.fori_loop(0, GRP, fire, 0)
        pltpu.make_async_copy(t3.at[wid, pl.ds(0, GRP)], val_v, sem).wait()
        return carry

    lax.fori_loop(0, NGRP, grp, 0)
    plsc.subcore_barrier()

    def cgrp(g, carry):
        def fire2(j, c2):
            pltpu.async_copy(
                inv_sh.at[idx_v.at[cid * R1 + g * CG + j]], val_v.at[j], sem)
            return c2

        lax.fori_loop(0, CG, fire2, 0)
        pltpu.make_async_copy(t3.at[wid, pl.ds(0, CG)],
                              val_v.at[pl.ds(0, CG)], sem).wait()
        pltpu.sync_copy(val_v.at[pl.ds(0, CG)], t3.at[wid, pl.ds(g * CG, CG)])
        return carry

    lax.fori_loop(0, NCG, cgrp, 0)


@functools.partial(
    pl.kernel,
    mesh=_mesh,
    out_type=jax.ShapeDtypeStruct((NW, XR, 128), jnp.int32),
    scratch_types=[
        pltpu.VMEM((RC, 128), jnp.int32),
        pltpu.VMEM((RC, 128), jnp.int32),
        pltpu.VMEM_SHARED((PV,), jnp.int32),
        pltpu.SemaphoreType.DMA,
    ],
)
def _lookup_kernel(x3, t_flat, out3, x_v, o_v, t_sh, sem):
    sid = lax.axis_index("s")
    wid = sid * NC + lax.axis_index("c")

    pltpu.sync_copy(t_flat.at[pl.ds(sid * SPC, SPC)],
                    t_sh.at[pl.ds(sid * SPC, SPC)])
    plsc.subcore_barrier()

    for ch in range(NCH):
        pltpu.sync_copy(x3.at[wid, pl.ds(ch * RC, RC)], x_v)

        def g1(j, carry):
            pltpu.async_copy(t_sh.at[x_v.at[j]], o_v.at[j], sem)
            return carry

        lax.fori_loop(0, RC, g1, 0)
        pltpu.make_async_copy(x3.at[wid, pl.ds(0, RC)], o_v, sem).wait()

        pltpu.sync_copy(o_v, out3.at[wid, pl.ds(ch * RC, RC)])


def kernel(x, input_perm):
    pad = jnp.arange(V, PV, dtype=jnp.int32)
    perm_p = jnp.concatenate([input_perm.astype(jnp.int32), pad])
    t3 = _build_kernel(perm_p.reshape(NS * RS, 128))
    x3 = x.reshape(NW, XR, 128)
    out3 = _lookup_kernel(x3, t3.reshape(PV))
    return out3.reshape(x.shape)


# speedup vs baseline: 371.8101x; 1.0900x over previous
import functools
import jax
import jax.numpy as jnp
from jax import lax
from jax.experimental import pallas as pl
from jax.experimental.pallas import tpu as pltpu
from jax.experimental.pallas import tpu_sc as plsc

NC = 2
NS = 16
NW = NC * NS

V = 1_000_000
C1 = 31_744
R1 = C1 // 128
PV = NW * C1
SPC = PV // NS
RS = SPC // 128
GRP = 16
NGRP = RS // GRP
CG = 8
NCG = R1 // CG

XR = 800
RC = 80
NCH = XR // RC

_mesh = plsc.VectorSubcoreMesh(core_axis_name="c", subcore_axis_name="s")


@functools.partial(
    pl.kernel,
    mesh=_mesh,
    out_type=jax.ShapeDtypeStruct((NW, R1, 128), jnp.int32),
    scratch_types=[
        pltpu.VMEM((RS, 128), jnp.int32),
        pltpu.VMEM((GRP, 128), jnp.int32),
        pltpu.VMEM_SHARED((PV,), jnp.int32),
        pltpu.SemaphoreType.DMA,
    ],
)
def _build_kernel(perm2d, t3, idx_v, val_v, inv_sh, sem):
    cid = lax.axis_index("c")
    sid = lax.axis_index("s")
    wid = sid * NC + cid
    base = sid * SPC
    lane = lax.iota(jnp.int32, 16)

    pltpu.sync_copy(perm2d.at[pl.ds(sid * RS, RS)], idx_v)

    def grp(g, carry):
        def mkval(t, c2):
            val_v[t // 8, pl.ds((t % 8) * 16, 16)] = lane + (
                base + g * (GRP * 128) + t * 16)
            return c2

        lax.fori_loop(0, GRP * 8, mkval, 0)

        def fire(j, c2):
            pltpu.async_copy(val_v.at[j], inv_sh.at[idx_v.at[g * GRP + j]],
                             sem)
            return c2

        lax.fori_loop(0, GRP, fire, 0)
        pltpu.make_async_copy(t3.at[wid, pl.ds(0, GRP)], val_v, sem).wait()
        return carry

    lax.fori_loop(0, NGRP, grp, 0)
    plsc.subcore_barrier()

    def cgrp(g, carry):
        def fire2(j, c2):
            pltpu.async_copy(
                inv_sh.at[idx_v.at[cid * R1 + g * CG + j]], val_v.at[j], sem)
            return c2

        lax.fori_loop(0, CG, fire2, 0)
        pltpu.make_async_copy(t3.at[wid, pl.ds(0, CG)],
                              val_v.at[pl.ds(0, CG)], sem).wait()
        pltpu.sync_copy(val_v.at[pl.ds(0, CG)], t3.at[wid, pl.ds(g * CG, CG)])
        return carry

    lax.fori_loop(0, NCG, cgrp, 0)


@functools.partial(
    pl.kernel,
    mesh=_mesh,
    out_type=jax.ShapeDtypeStruct((NW, XR, 128), jnp.int32),
    scratch_types=[
        pltpu.VMEM((2, RC, 128), jnp.int32),
        pltpu.VMEM((2, RC, 128), jnp.int32),
        pltpu.VMEM_SHARED((PV,), jnp.int32),
        pltpu.SemaphoreType.DMA,
        pltpu.SemaphoreType.DMA,
        pltpu.SemaphoreType.DMA,
    ],
)
def _lookup_kernel(x3, t_flat, out3, x_v, o_v, t_sh, sem_g, sem_ld, sem_st):
    sid = lax.axis_index("s")
    wid = sid * NC + lax.axis_index("c")

    pltpu.async_copy(x3.at[wid, pl.ds(0, RC)], x_v.at[0], sem_ld)
    pltpu.sync_copy(t_flat.at[pl.ds(sid * SPC, SPC)],
                    t_sh.at[pl.ds(sid * SPC, SPC)])
    plsc.subcore_barrier()

    for ch in range(NCH):
        b = ch % 2
        pltpu.make_async_copy(x3.at[wid, pl.ds(0, RC)], x_v.at[b],
                              sem_ld).wait()
        if ch + 1 < NCH:
            pltpu.async_copy(x3.at[wid, pl.ds((ch + 1) * RC, RC)],
                             x_v.at[1 - b], sem_ld)
        if ch >= 2:
            pltpu.make_async_copy(x3.at[wid, pl.ds(0, RC)], o_v.at[b],
                                  sem_st).wait()

        def g1(j, carry):
            pltpu.async_copy(t_sh.at[x_v.at[b, j]], o_v.at[b, j], sem_g)
            return carry

        lax.fori_loop(0, RC, g1, 0)
        pltpu.make_async_copy(x3.at[wid, pl.ds(0, RC)], o_v.at[b],
                              sem_g).wait()
        pltpu.async_copy(o_v.at[b], out3.at[wid, pl.ds(ch * RC, RC)], sem_st)

    pltpu.make_async_copy(x3.at[wid, pl.ds(0, RC)], o_v.at[0], sem_st).wait()
    pltpu.make_async_copy(x3.at[wid, pl.ds(0, RC)], o_v.at[1], sem_st).wait()


def kernel(x, input_perm):
    pad = jnp.arange(V, PV, dtype=jnp.int32)
    perm_p = jnp.concatenate([input_perm.astype(jnp.int32), pad])
    t3 = _build_kernel(perm_p.reshape(NS * RS, 128))
    x3 = x.reshape(NW, XR, 128)
    out3 = _lookup_kernel(x3, t3.reshape(PV))
    return out3.reshape(x.shape)


# speedup vs baseline: 410.7584x; 1.1048x over previous
import functools
import jax
import jax.numpy as jnp
from jax import lax
from jax.experimental import pallas as pl
from jax.experimental.pallas import tpu as pltpu
from jax.experimental.pallas import tpu_sc as plsc

NC = 2
NS = 16
NW = NC * NS

V = 1_000_000
C1 = 31_744
R1 = C1 // 128
PV = NW * C1
SPC = PV // NS
RS = SPC // 128
GRP = 16
SG = 8
NSG = RS // SG
CG = 8
NCG = R1 // CG

XR = 800
RC = 80
NCH = XR // RC

_mesh = plsc.VectorSubcoreMesh(core_axis_name="c", subcore_axis_name="s")


@functools.partial(
    pl.kernel,
    mesh=_mesh,
    out_type=jax.ShapeDtypeStruct((NW, R1, 128), jnp.int32),
    scratch_types=[
        pltpu.VMEM((RS, 128), jnp.int32),
        pltpu.VMEM((GRP, 128), jnp.int32),
        pltpu.VMEM_SHARED((PV,), jnp.int32),
        pltpu.SemaphoreType.DMA,
        pltpu.SemaphoreType.DMA,
    ],
)
def _build_kernel(perm2d, t3, idx_v, val_v, inv_sh, sem, sem_w):
    cid = lax.axis_index("c")
    sid = lax.axis_index("s")
    wid = sid * NC + cid
    base = sid * SPC
    lane = lax.iota(jnp.int32, 16)

    pltpu.sync_copy(perm2d.at[pl.ds(sid * RS, RS)], idx_v)

    def build_fire(g, h):
        def mkval(t, c2):
            val_v[h + t // 8, pl.ds((t % 8) * 16, 16)] = lane + (
                base + g * (SG * 128) + t * 16)
            return c2

        lax.fori_loop(0, SG * 8, mkval, 0)

        def fire(j, c2):
            pltpu.async_copy(val_v.at[h + j], inv_sh.at[idx_v.at[g * SG + j]],
                             sem)
            return c2

        lax.fori_loop(0, SG, fire, 0)

    build_fire(0, 0)
    build_fire(1, SG)

    def grp(g, carry):
        pltpu.make_async_copy(t3.at[wid, pl.ds(0, SG)],
                              val_v.at[pl.ds(0, SG)], sem).wait()
        build_fire(g, (g % 2) * SG)
        return carry

    lax.fori_loop(2, NSG, grp, 0)
    pltpu.make_async_copy(t3.at[wid, pl.ds(0, GRP)], val_v, sem).wait()
    plsc.subcore_barrier()

    def cfire(g, h):
        def fire2(j, c2):
            pltpu.async_copy(
                inv_sh.at[idx_v.at[cid * R1 + g * CG + j]],
                val_v.at[h + j], sem)
            return c2

        lax.fori_loop(0, CG, fire2, 0)

    def cdrain_write(g, h):
        pltpu.make_async_copy(t3.at[wid, pl.ds(0, CG)],
                              val_v.at[pl.ds(0, CG)], sem).wait()
        pltpu.async_copy(val_v.at[pl.ds(h, CG)],
                         t3.at[wid, pl.ds(g * CG, CG)], sem_w)

    cfire(0, 0)
    cfire(1, CG)
    cdrain_write(0, 0)

    def cgrp(g, carry):
        pltpu.make_async_copy(t3.at[wid, pl.ds(0, CG)],
                              val_v.at[pl.ds(0, CG)], sem_w).wait()
        cfire(g, (g % 2) * CG)
        cdrain_write(g - 1, ((g - 1) % 2) * CG)
        return carry

    lax.fori_loop(2, NCG, cgrp, 0)
    cdrain_write(NCG - 1, ((NCG - 1) % 2) * CG)
    pltpu.make_async_copy(t3.at[wid, pl.ds(0, GRP)], val_v, sem_w).wait()


@functools.partial(
    pl.kernel,
    mesh=_mesh,
    out_type=jax.ShapeDtypeStruct((NW, XR, 128), jnp.int32),
    scratch_types=[
        pltpu.VMEM((2, RC, 128), jnp.int32),
        pltpu.VMEM((2, RC, 128), jnp.int32),
        pltpu.VMEM_SHARED((PV,), jnp.int32),
        pltpu.SemaphoreType.DMA,
        pltpu.SemaphoreType.DMA,
        pltpu.SemaphoreType.DMA,
    ],
)
def _lookup_kernel(x3, t_flat, out3, x_v, o_v, t_sh, sem_g, sem_ld, sem_st):
    sid = lax.axis_index("s")
    wid = sid * NC + lax.axis_index("c")

    pltpu.async_copy(x3.at[wid, pl.ds(0, RC)], x_v.at[0], sem_ld)
    pltpu.sync_copy(t_flat.at[pl.ds(sid * SPC, SPC)],
                    t_sh.at[pl.ds(sid * SPC, SPC)])
    plsc.subcore_barrier()

    for ch in range(NCH):
        b = ch % 2
        pltpu.make_async_copy(x3.at[wid, pl.ds(0, RC)], x_v.at[b],
                              sem_ld).wait()
        if ch + 1 < NCH:
            pltpu.async_copy(x3.at[wid, pl.ds((ch + 1) * RC, RC)],
                             x_v.at[1 - b], sem_ld)
        if ch >= 2:
            pltpu.make_async_copy(x3.at[wid, pl.ds(0, RC)], o_v.at[b],
                                  sem_st).wait()

        def g1(j, carry):
            pltpu.async_copy(t_sh.at[x_v.at[b, j]], o_v.at[b, j], sem_g)
            return carry

        lax.fori_loop(0, RC, g1, 0)
        pltpu.make_async_copy(x3.at[wid, pl.ds(0, RC)], o_v.at[b],
                              sem_g).wait()
        pltpu.async_copy(o_v.at[b], out3.at[wid, pl.ds(ch * RC, RC)], sem_st)

    pltpu.make_async_copy(x3.at[wid, pl.ds(0, RC)], o_v.at[0], sem_st).wait()
    pltpu.make_async_copy(x3.at[wid, pl.ds(0, RC)], o_v.at[1], sem_st).wait()


def kernel(x, input_perm):
    pad = jnp.arange(V, PV, dtype=jnp.int32)
    perm_p = jnp.concatenate([input_perm.astype(jnp.int32), pad])
    t3 = _build_kernel(perm_p.reshape(NS * RS, 128))
    x3 = x.reshape(NW, XR, 128)
    out3 = _lookup_kernel(x3, t3.reshape(PV))
    return out3.reshape(x.shape)
